# baseline (speedup 1.0000x reference)
import functools

import jax
import jax.numpy as jnp
from jax import lax
from jax.experimental import pallas as pl
from jax.experimental.pallas import tpu as pltpu
from jax.experimental.pallas import tpu_sc as plsc

B = 16384
SL = 73
DC, DR, DB = 16, 8, 6
VC, VR, VB = 101, 9, 51

_info = plsc.get_sparse_core_info()
_NC, _NS, _L = 1, _info.num_subcores, _info.num_lanes
NW = _NC * _NS
BPW = B // NW


def _sc_body(xf_hbm, wc_hbm, oc_hbm,
             idxc_v, colc_v, rc_v, wc_t, wc_s, sem_e, sem_g, sem_o):
    sid = lax.axis_index("s")
    wid = sid * _NC + lax.axis_index("c")
    base = wid * BPW

    lanes = lax.iota(jnp.int32, _L)
    for g in range(BPW // _L):
        s = pl.ds(g * _L, _L)
        idxc_v[s] = (base + g * _L + lanes) * SL

    @pl.when(sid == 0)
    def _stage_table():
        pltpu.sync_copy(wc_hbm, wc_t)
        pltpu.sync_copy(wc_t, wc_s)

    pltpu.async_copy(xf_hbm.at[idxc_v], colc_v, sem_e).wait()
    for g in range(BPW // _L):
        s = pl.ds(g * _L, _L)
        idxc_v[s] = colc_v[s].astype(jnp.int32)

    plsc.subcore_barrier()

    pltpu.async_copy(wc_s.at[idxc_v], rc_v, sem_g).wait()

    pltpu.async_copy(rc_v, oc_hbm.at[pl.ds(base, BPW)], sem_o).wait()


_sc_call = functools.partial(
    pl.kernel,
    mesh=plsc.VectorSubcoreMesh(core_axis_name="c", subcore_axis_name="s",
                                num_cores=_NC),
    compiler_params=pltpu.CompilerParams(use_tc_tiling_on_sc=False),
    out_type=jax.ShapeDtypeStruct((B, DC), jnp.float32),
    scratch_types=[
        pltpu.VMEM((BPW,), jnp.int32),
        pltpu.VMEM((BPW,), jnp.float32),
        pltpu.VMEM((BPW, DC), jnp.float32),
        pltpu.VMEM((VC, DC), jnp.float32),
        pltpu.VMEM_SHARED((VC, DC), jnp.float32),
        pltpu.SemaphoreType.DMA,
        pltpu.SemaphoreType.DMA,
        pltpu.SemaphoreType.DMA,
    ],
)(_sc_body)


_RB = 2048


def _tc_body(x_ref, wr_ref, wb_ref, os_ref, orr_ref, ob_ref):
    blk = x_ref[...]
    os_ref[...] = blk[:, 3:]
    role = blk[:, 1:2].astype(jnp.int32)
    oh_r = (role == lax.broadcasted_iota(jnp.int32, (_RB, VR), 1))
    orr_ref[...] = jnp.dot(oh_r.astype(jnp.float32), wr_ref[...],
                           preferred_element_type=jnp.float32)
    buff = blk[:, 2:3].astype(jnp.int32)
    oh_b = (buff == lax.broadcasted_iota(jnp.int32, (_RB, VB), 1))
    ob_ref[...] = jnp.dot(oh_b.astype(jnp.float32), wb_ref[...],
                          preferred_element_type=jnp.float32)


_tc_call = pl.pallas_call(
    _tc_body,
    grid=(B // _RB,),
    in_specs=[
        pl.BlockSpec((_RB, SL), lambda i: (i, 0)),
        pl.BlockSpec((VR, DR), lambda i: (0, 0)),
        pl.BlockSpec((VB, DB), lambda i: (0, 0)),
    ],
    out_specs=(
        pl.BlockSpec((_RB, SL - 3), lambda i: (i, 0)),
        pl.BlockSpec((_RB, DR), lambda i: (i, 0)),
        pl.BlockSpec((_RB, DB), lambda i: (i, 0)),
    ),
    out_shape=(
        jax.ShapeDtypeStruct((B, SL - 3), jnp.float32),
        jax.ShapeDtypeStruct((B, DR), jnp.float32),
        jax.ShapeDtypeStruct((B, DB), jnp.float32),
    ),
)


def kernel(x, W_char, W_role, W_buff):
    oc = _sc_call(x.reshape(-1), W_char)
    os, orr, ob = _tc_call(x, W_role, W_buff)
    return oc, orr, ob, os


# speedup vs baseline: 1.1018x; 1.1018x over previous
import functools

import jax
import jax.numpy as jnp
from jax import lax
from jax.experimental import pallas as pl
from jax.experimental.pallas import tpu as pltpu
from jax.experimental.pallas import tpu_sc as plsc

B = 16384
SL = 73
DC, DR, DB = 16, 8, 6
VC, VR, VB = 101, 9, 51

_info = plsc.get_sparse_core_info()
_NC, _NS, _L = 1, _info.num_subcores, _info.num_lanes
NW = _NC * _NS
BPW = B // NW


def _sc_body(ic_hbm, wc_hbm, oc_hbm,
             idxc_v, rc_v, wc_t, wc_s, sem_e, sem_g, sem_o):
    sid = lax.axis_index("s")
    wid = sid * _NC + lax.axis_index("c")
    base = wid * BPW

    i1 = pltpu.async_copy(ic_hbm.at[pl.ds(base, BPW)], idxc_v, sem_e)

    @pl.when(sid == 0)
    def _stage_table():
        pltpu.sync_copy(wc_hbm, wc_t)
        pltpu.sync_copy(wc_t, wc_s)

    plsc.subcore_barrier()
    i1.wait()

    pltpu.async_copy(wc_s.at[idxc_v], rc_v, sem_g).wait()

    pltpu.async_copy(rc_v, oc_hbm.at[pl.ds(base, BPW)], sem_o).wait()


_sc_call = functools.partial(
    pl.kernel,
    mesh=plsc.VectorSubcoreMesh(core_axis_name="c", subcore_axis_name="s",
                                num_cores=_NC),
    compiler_params=pltpu.CompilerParams(use_tc_tiling_on_sc=False),
    out_type=jax.ShapeDtypeStruct((B, DC), jnp.float32),
    scratch_types=[
        pltpu.VMEM((BPW,), jnp.int32),
        pltpu.VMEM((BPW, DC), jnp.float32),
        pltpu.VMEM((VC, DC), jnp.float32),
        pltpu.VMEM_SHARED((VC, DC), jnp.float32),
        pltpu.SemaphoreType.DMA,
        pltpu.SemaphoreType.DMA,
        pltpu.SemaphoreType.DMA,
    ],
)(_sc_body)


_RB = 2048


def _tc_body(x_ref, wr_ref, wb_ref, ic_ref, os_ref, orr_ref, ob_ref):
    blk = x_ref[...]
    ic_ref[...] = blk[:, 0].astype(jnp.int32)
    os_ref[...] = blk[:, 3:]
    role = blk[:, 1:2].astype(jnp.int32)
    oh_r = (role == lax.broadcasted_iota(jnp.int32, (_RB, VR), 1))
    orr_ref[...] = jnp.dot(oh_r.astype(jnp.float32), wr_ref[...],
                           preferred_element_type=jnp.float32)
    buff = blk[:, 2:3].astype(jnp.int32)
    oh_b = (buff == lax.broadcasted_iota(jnp.int32, (_RB, VB), 1))
    ob_ref[...] = jnp.dot(oh_b.astype(jnp.float32), wb_ref[...],
                          preferred_element_type=jnp.float32)


_tc_call = pl.pallas_call(
    _tc_body,
    grid=(B // _RB,),
    in_specs=[
        pl.BlockSpec((_RB, SL), lambda i: (i, 0)),
        pl.BlockSpec((VR, DR), lambda i: (0, 0)),
        pl.BlockSpec((VB, DB), lambda i: (0, 0)),
    ],
    out_specs=(
        pl.BlockSpec((_RB,), lambda i: (i,)),
        pl.BlockSpec((_RB, SL - 3), lambda i: (i, 0)),
        pl.BlockSpec((_RB, DR), lambda i: (i, 0)),
        pl.BlockSpec((_RB, DB), lambda i: (i, 0)),
    ),
    out_shape=(
        jax.ShapeDtypeStruct((B,), jnp.int32),
        jax.ShapeDtypeStruct((B, SL - 3), jnp.float32),
        jax.ShapeDtypeStruct((B, DR), jnp.float32),
        jax.ShapeDtypeStruct((B, DB), jnp.float32),
    ),
)


def kernel(x, W_char, W_role, W_buff):
    ic, os, orr, ob = _tc_call(x, W_role, W_buff)
    oc = _sc_call(ic, W_char)
    return oc, orr, ob, os
